# Initial kernel scaffold; baseline (speedup 1.0000x reference)
import jax
import jax.numpy as jnp
from jax.experimental import pallas as pl


def kernel(h, coords, a, edge_index, w_e1, b_e1, w_e2, b_e2, w_att, b_att, w_n1, b_n1, w_n2, b_n2, w_c1, b_c1, w_c2, b_c2, w_c3):
    raise NotImplementedError("write your pallas kernel here")


# speedup vs baseline: 2.6838x; 2.6838x over previous
import functools

import jax
import jax.numpy as jnp
from jax import lax
from jax.experimental import pallas as pl
from jax.experimental.pallas import tpu as pltpu
from jax.experimental.pallas import tpu_sc as plsc

N = 10000
E = 320000
H = 128
EF = 16
DIM = 3
XP = 16
TW = H + XP

NC = 2
NS = 16
NW = NC * NS
EPW = E // NW
CH = 400
NCHUNK = EPW // CH
CH3 = 200
NCHUNK3 = EPW // CH3

TE = 2000
TN = 2000


def _k1_body(tab, src, dst, hs_out, hd_out, sidx_v, didx_v, hs_v, hd_v, sem):
    wid = lax.axis_index("s") * NC + lax.axis_index("c")

    def chunk(j, carry):
        base = pl.multiple_of(wid * EPW + j * CH, 8)
        pltpu.sync_copy(src.at[pl.ds(base, CH)], sidx_v)
        pltpu.sync_copy(dst.at[pl.ds(base, CH)], didx_v)
        cp1 = pltpu.async_copy(tab.at[sidx_v], hs_v, sem)
        cp2 = pltpu.async_copy(tab.at[didx_v], hd_v, sem)
        cp1.wait()
        cp2.wait()
        pltpu.sync_copy(hs_v, hs_out.at[pl.ds(base, CH)])
        pltpu.sync_copy(hd_v, hd_out.at[pl.ds(base, CH)])
        return carry

    lax.fori_loop(0, NCHUNK, chunk, 0)


def _silu(x):
    return x * jax.nn.sigmoid(x)


def _k2_body(hx_s, hx_d, a,
             we1s, we1d, we1r, we1a, be1, we2, be2, watt_t, batt,
             wc1s, wc1d, wc1r, wc1a, bc1, wc2, bc2, wc3_t,
             msg_h_out, msg_x_out):
    hs = hx_s[:, :H]
    xs = hx_s[:, H:TW]
    hd = hx_d[:, :H]
    xd = hx_d[:, H:TW]
    av = a[...]

    diffs = xs - xd
    rad = jnp.sum(diffs * diffs, axis=1, keepdims=True)

    t = _silu(hs @ wc1s[...] + hd @ wc1d[...] + rad * wc1r[...]
              + av @ wc1a[...] + bc1[...])
    t = _silu(t @ wc2[...] + bc2[...])
    scale = jnp.sum(t * wc3_t[...], axis=1, keepdims=True)
    inv = 1.0 / (jnp.sqrt(rad + 1e-05) + 1.0)
    msg_x_out[...] = scale * inv * diffs

    mh = _silu(hs @ we1s[...] + hd @ we1d[...] + rad * we1r[...]
               + av @ we1a[...] + be1[...])
    mh = _silu(mh @ we2[...] + be2[...])
    att = jax.nn.sigmoid(jnp.sum(mh * watt_t[...], axis=1, keepdims=True)
                         + batt[...])
    msg_h_out[...] = att * mh


def _k2_edge_mlp(hx_s, hx_d, a, wparams):
    (we1s, we1d, we1r, we1a, be1, we2, be2, watt_t, batt,
     wc1s, wc1d, wc1r, wc1a, bc1, wc2, bc2, wc3_t) = wparams
    grid = (E // TE,)
    full = lambda shape: pl.BlockSpec(shape, lambda i: (0, 0))
    return pl.pallas_call(
        _k2_body,
        grid=grid,
        in_specs=[
            pl.BlockSpec((TE, TW), lambda i: (i, 0)),
            pl.BlockSpec((TE, TW), lambda i: (i, 0)),
            pl.BlockSpec((TE, EF), lambda i: (i, 0)),
            full((H, H)), full((H, H)), full((1, H)), full((EF, H)),
            full((1, H)), full((H, H)), full((1, H)), full((1, H)),
            full((1, 1)),
            full((H, H)), full((H, H)), full((1, H)), full((EF, H)),
            full((1, H)), full((H, H)), full((1, H)), full((1, H)),
        ],
        out_specs=[
            pl.BlockSpec((TE, H), lambda i: (i, 0)),
            pl.BlockSpec((TE, XP), lambda i: (i, 0)),
        ],
        out_shape=[
            jax.ShapeDtypeStruct((E, H), jnp.float32),
            jax.ShapeDtypeStruct((E, XP), jnp.float32),
        ],
    )(hx_s, hx_d, a, we1s, we1d, we1r, we1a, be1, we2, be2, watt_t, batt,
      wc1s, wc1d, wc1r, wc1a, bc1, wc2, bc2, wc3_t)


_RPT = N // NS


def _k3_body(msg_h, msg_x, dst, zeros_h, zeros_x, out_h, out_x,
             idx_v, mh_v, mx_v, acc_h, acc_x, sem):
    c = lax.axis_index("c")
    s = lax.axis_index("s")
    wid = s * NC + c
    r0 = s * _RPT

    pltpu.sync_copy(zeros_h.at[pl.ds(r0, _RPT)], acc_h.at[pl.ds(r0, _RPT)])
    pltpu.sync_copy(zeros_x.at[pl.ds(r0, _RPT)], acc_x.at[pl.ds(r0, _RPT)])
    plsc.subcore_barrier()

    def chunk(j, carry):
        base = pl.multiple_of(wid * EPW + j * CH3, 8)
        pltpu.sync_copy(dst.at[pl.ds(base, CH3)], idx_v)
        cp1 = pltpu.async_copy(msg_h.at[pl.ds(base, CH3)], mh_v, sem)
        cp2 = pltpu.async_copy(msg_x.at[pl.ds(base, CH3)], mx_v, sem)
        cp1.wait()
        cp2.wait()
        pltpu.sync_copy(mh_v, acc_h.at[idx_v], add=True)
        pltpu.sync_copy(mx_v, acc_x.at[idx_v], add=True)
        return carry

    lax.fori_loop(0, NCHUNK3, chunk, 0)
    plsc.subcore_barrier()

    pltpu.sync_copy(acc_h.at[pl.ds(r0, _RPT)], out_h.at[c, pl.ds(r0, _RPT)])
    pltpu.sync_copy(acc_x.at[pl.ds(r0, _RPT)], out_x.at[c, pl.ds(r0, _RPT)])


def _k4_body(h, p0, p1, coords, x0, x1, wn1h, wn1g, bn1, wn2, bn2,
             h_out, c_out):
    hb = h[...]
    hagg = p0[...] + p1[...]
    nh = _silu(hb @ wn1h[...] + hagg @ wn1g[...] + bn1[...])
    h_out[...] = hb + nh @ wn2[...] + bn2[...]
    c_out[...] = coords[...] + (x0[...] + x1[...])[:, :DIM]


def _k4_node_mlp(h, p0, p1, coords, x0, x1, wn1h, wn1g, bn1, wn2, bn2):
    grid = (N // TN,)
    full = lambda shape: pl.BlockSpec(shape, lambda i: (0, 0))
    return pl.pallas_call(
        _k4_body,
        grid=grid,
        in_specs=[
            pl.BlockSpec((TN, H), lambda i: (i, 0)),
            pl.BlockSpec((TN, H), lambda i: (i, 0)),
            pl.BlockSpec((TN, H), lambda i: (i, 0)),
            pl.BlockSpec((TN, DIM), lambda i: (i, 0)),
            pl.BlockSpec((TN, XP), lambda i: (i, 0)),
            pl.BlockSpec((TN, XP), lambda i: (i, 0)),
            full((H, H)), full((H, H)), full((1, H)), full((H, H)),
            full((1, H)),
        ],
        out_specs=[
            pl.BlockSpec((TN, H), lambda i: (i, 0)),
            pl.BlockSpec((TN, DIM), lambda i: (i, 0)),
        ],
        out_shape=[
            jax.ShapeDtypeStruct((N, H), jnp.float32),
            jax.ShapeDtypeStruct((N, DIM), jnp.float32),
        ],
    )(h, p0, p1, coords, x0, x1, wn1h, wn1g, bn1, wn2, bn2)


@functools.lru_cache(maxsize=None)
def _get_sc_kernels():
    mesh = plsc.VectorSubcoreMesh(core_axis_name="c", subcore_axis_name="s")
    params = pltpu.CompilerParams(use_tc_tiling_on_sc=False)
    k1 = pl.kernel(
        _k1_body,
        out_type=[
            jax.ShapeDtypeStruct((E, TW), jnp.float32),
            jax.ShapeDtypeStruct((E, TW), jnp.float32),
        ],
        mesh=mesh,
        scratch_types=[
            pltpu.VMEM((CH,), jnp.int32),
            pltpu.VMEM((CH,), jnp.int32),
            pltpu.VMEM((CH, TW), jnp.float32),
            pltpu.VMEM((CH, TW), jnp.float32),
            pltpu.SemaphoreType.DMA,
        ],
        compiler_params=params,
    )
    k3 = pl.kernel(
        _k3_body,
        out_type=[
            jax.ShapeDtypeStruct((NC, N, H), jnp.float32),
            jax.ShapeDtypeStruct((NC, N, XP), jnp.float32),
        ],
        mesh=mesh,
        scratch_types=[
            pltpu.VMEM((CH3,), jnp.int32),
            pltpu.VMEM((CH3, H), jnp.float32),
            pltpu.VMEM((CH3, XP), jnp.float32),
            pltpu.VMEM_SHARED((N, H), jnp.float32),
            pltpu.VMEM_SHARED((N, XP), jnp.float32),
            pltpu.SemaphoreType.DMA,
        ],
        compiler_params=params,
    )
    return k1, k3


def kernel(h, coords, a, edge_index, w_e1, b_e1, w_e2, b_e2, w_att, b_att,
           w_n1, b_n1, w_n2, b_n2, w_c1, b_c1, w_c2, b_c2, w_c3):
    coords_p = jnp.pad(coords, ((0, 0), (0, XP - DIM)))
    tab = jnp.concatenate([h, coords_p], axis=1)
    src = edge_index[0]
    dst = edge_index[1]

    _k1_gather, _k3_scatter = _get_sc_kernels()
    hx_s, hx_d = _k1_gather(tab, src, dst)

    wparams = (
        w_e1[:H], w_e1[H:2 * H], w_e1[2 * H:2 * H + 1], w_e1[2 * H + 1:],
        b_e1.reshape(1, H), w_e2, b_e2.reshape(1, H),
        w_att.reshape(1, H), b_att.reshape(1, 1),
        w_c1[:H], w_c1[H:2 * H], w_c1[2 * H:2 * H + 1], w_c1[2 * H + 1:],
        b_c1.reshape(1, H), w_c2, b_c2.reshape(1, H),
        w_c3.reshape(1, H),
    )
    msg_h, msg_x = _k2_edge_mlp(hx_s, hx_d, a, wparams)

    zeros_h = jnp.zeros((N, H), jnp.float32)
    zeros_x = jnp.zeros((N, XP), jnp.float32)
    part_h, part_x = _k3_scatter(msg_h, msg_x, dst, zeros_h, zeros_x)

    h_out, coords_out = _k4_node_mlp(
        h, part_h[0], part_h[1], coords, part_x[0], part_x[1],
        w_n1[:H], w_n1[H:], b_n1.reshape(1, H), w_n2, b_n2.reshape(1, H))
    return (h_out, coords_out)
